# baseline (speedup 1.0000x reference)
import jax
import jax.numpy as jnp
from jax import lax
from jax.experimental import pallas as pl
from jax.experimental.pallas import tpu as pltpu
from jax.experimental.pallas import tpu_sc as plsc

_ROWS = 96
_COLS = 16384

_INFO = plsc.get_sparse_core_info()
_NC = _INFO.num_cores
_NS = _INFO.num_subcores
_NW = _NC * _NS
_COLS_PER_W = _COLS // _NW


def _add1_worker(x_hbm, o_hbm, buf, sem):
    wid = lax.axis_index("s") * _NC + lax.axis_index("c")
    base = wid * _COLS_PER_W
    pltpu.async_copy(x_hbm.at[:, pl.ds(base, _COLS_PER_W)], buf, sem).wait()

    @pl.loop(0, _ROWS)
    def _row(r):
        @pl.loop(0, _COLS_PER_W, step=16, unroll=8)
        def _add(o):
            buf[r, pl.ds(o, 16)] = buf[r, pl.ds(o, 16)] + 1.0

    pltpu.async_copy(buf, o_hbm.at[:, pl.ds(base, _COLS_PER_W)], sem).wait()


def kernel(input_xyzs):
    b, n, c = input_xyzs.shape
    x = jnp.transpose(input_xyzs, (2, 0, 1)).reshape(c * b, n)
    sc_kernel = pl.kernel(
        _add1_worker,
        out_type=jax.ShapeDtypeStruct((c * b, n), jnp.float32),
        mesh=plsc.VectorSubcoreMesh(core_axis_name="c", subcore_axis_name="s"),
        scratch_types=[
            pltpu.VMEM((_ROWS, _COLS_PER_W), jnp.float32),
            pltpu.SemaphoreType.DMA,
        ],
    )
    out = sc_kernel(x)
    return jnp.transpose(out.reshape(c, b, n), (1, 2, 0))


# speedup vs baseline: 5.3170x; 5.3170x over previous
import jax
import jax.numpy as jnp
from jax.experimental import pallas as pl
from jax.experimental.pallas import tpu as pltpu

_ROWS = 96
_COLS = 16384
_BLOCK_ROWS = 48


def _add1_block(x_ref, o_ref):
    o_ref[...] = x_ref[...] + 1.0


def _outer(x_hbm, o_hbm):
    pltpu.emit_pipeline(
        _add1_block,
        grid=(_ROWS // _BLOCK_ROWS,),
        in_specs=[
            pl.BlockSpec(
                (_BLOCK_ROWS, _COLS),
                lambda i: (i, 0),
                pipeline_mode=pl.Buffered(buffer_count=2),
            )
        ],
        out_specs=[
            pl.BlockSpec(
                (_BLOCK_ROWS, _COLS),
                lambda i: (i, 0),
                pipeline_mode=pl.Buffered(buffer_count=2),
            )
        ],
    )(x_hbm, o_hbm)


def kernel(input_xyzs):
    b, n, c = input_xyzs.shape
    x = jnp.transpose(input_xyzs, (2, 0, 1)).reshape(c * b, n)
    out = pl.pallas_call(
        _outer,
        out_shape=jax.ShapeDtypeStruct((c * b, n), jnp.float32),
        in_specs=[pl.BlockSpec(memory_space=pl.ANY)],
        out_specs=pl.BlockSpec(memory_space=pl.ANY),
    )(x)
    return jnp.transpose(out.reshape(c, b, n), (1, 2, 0))
